# Initial kernel scaffold; baseline (speedup 1.0000x reference)
import functools

import jax
import jax.numpy as jnp
from jax import lax
from jax.experimental import pallas as pl
from jax.experimental.pallas import tpu as pltpu
from jax.experimental.pallas import tpu_sc as plsc

N = 10000
E = 320000
D_IN = 128
D_H = 256
D_E = 16
G = 128

C_CHUNK = 80
N_TILES = 16
ROWS_PER_TILE = N // N_TILES

f32 = jnp.float32


def _make_sc_edge(d_half):
    epw = E // N_TILES
    nch = epw // C_CHUNK
    npairs = nch // 2
    nvec = d_half // 16
    zrows = 125

    mesh = plsc.VectorSubcoreMesh(core_axis_name="c", subcore_axis_name="s")

    @functools.partial(
        pl.kernel,
        mesh=mesh,
        out_type=(
            jax.ShapeDtypeStruct((N, d_half), f32),
            jax.ShapeDtypeStruct((N, d_half), f32),
        ),
        scratch_types=[
            pltpu.VMEM((nch, C_CHUNK), jnp.int32),
            pltpu.VMEM((nch, C_CHUNK), jnp.int32),
            pltpu.VMEM((2, C_CHUNK, d_half), f32),
            pltpu.VMEM((2, C_CHUNK, d_half), f32),
            pltpu.VMEM((2, C_CHUNK, d_half), f32),
            pltpu.VMEM((125, d_half), f32),
            pltpu.VMEM_SHARED((N, d_half), f32),
            pltpu.SemaphoreType.DMA,
            pltpu.SemaphoreType.DMA,
            pltpu.SemaphoreType.DMA,
            pltpu.SemaphoreType.DMA,
            pltpu.SemaphoreType.DMA,
            pltpu.SemaphoreType.DMA,
        ],
    )
    def sc_edge(h0, h1, e0, e1, src2d, dst2d, out0, out1,
                sidx, didx, rows, ebuf, msg, zbuf, acc,
                sg0, sg1, se0, se1, ss0, ss1):
        cid = lax.axis_index("c")
        sid = lax.axis_index("s")
        sg = (sg0, sg1)
        se = (se0, se1)
        ss = (ss0, ss1)
        row0 = sid * nch

        def zfill(r, carry):
            for kv in range(nvec):
                zbuf[r, pl.ds(kv * 16, 16)] = jnp.zeros((16,), f32)
            return carry
        lax.fori_loop(0, zrows, zfill, 0)

        def run(h, e, out):
            pltpu.sync_copy(src2d.at[pl.ds(row0, nch)], sidx)
            pltpu.sync_copy(dst2d.at[pl.ds(row0, nch)], didx)

            for kz in range(ROWS_PER_TILE // zrows):
                pltpu.sync_copy(
                    zbuf, acc.at[pl.ds(sid * ROWS_PER_TILE + kz * zrows, zrows)])
            plsc.subcore_barrier()

            def gather_start(slot, j):
                pltpu.make_async_copy(
                    h.at[sidx.at[j]], rows.at[slot], sg[slot]).start()
                pltpu.make_async_copy(
                    e.at[pl.ds((row0 + j) * C_CHUNK, C_CHUNK)],
                    ebuf.at[slot], se[slot]).start()

            def gather_wait(slot, j):
                pltpu.make_async_copy(
                    h.at[sidx.at[j]], rows.at[slot], sg[slot]).wait()
                pltpu.make_async_copy(
                    e.at[pl.ds((row0 + j) * C_CHUNK, C_CHUNK)],
                    ebuf.at[slot], se[slot]).wait()

            def scatter_start(slot, j):
                pltpu.make_async_copy(
                    msg.at[slot], acc.at[didx.at[j]], ss[slot],
                    add=True).start()

            def scatter_wait(slot, j):
                pltpu.make_async_copy(
                    msg.at[slot], acc.at[didx.at[j]], ss[slot],
                    add=True).wait()

            def compute(slot):
                def body(r, carry):
                    for kv in range(nvec):
                        a = rows[slot, r, pl.ds(kv * 16, 16)]
                        bv = ebuf[slot, r, pl.ds(kv * 16, 16)]
                        msg[slot, r, pl.ds(kv * 16, 16)] = (
                            jnp.maximum(a + bv, 0.0))
                    return carry
                lax.fori_loop(0, C_CHUNK, body, 0)

            gather_start(0, 0)

            def pair(p, carry):
                j0 = 2 * p
                j1 = j0 + 1
                gather_start(1, j1)
                gather_wait(0, j0)

                @pl.when(p > 0)
                def _():
                    scatter_wait(0, j0 - 2)
                compute(0)
                scatter_start(0, j0)

                @pl.when(p < npairs - 1)
                def _():
                    gather_start(0, j0 + 2)
                gather_wait(1, j1)

                @pl.when(p > 0)
                def _():
                    scatter_wait(1, j1 - 2)
                compute(1)
                scatter_start(1, j1)
                return carry

            lax.fori_loop(0, npairs, pair, 0)
            scatter_wait(0, nch - 2)
            scatter_wait(1, nch - 1)
            plsc.subcore_barrier()

            pltpu.sync_copy(
                acc.at[pl.ds(sid * ROWS_PER_TILE, ROWS_PER_TILE)],
                out.at[pl.ds(sid * ROWS_PER_TILE, ROWS_PER_TILE)])

        @pl.when(cid == 0)
        def _():
            run(h0, e0, out0)

        @pl.when(cid == 1)
        def _():
            run(h1, e1, out1)

    return sc_edge


zrows = 125
_sc_edge_64 = _make_sc_edge(64)
_sc_edge_128 = _make_sc_edge(128)


_BE = 1000
_BN = 1000
_BP = 2000


def _dotT(a, w):
    return lax.dot_general(a, w, (((1,), (1,)), ((), ())),
                           preferred_element_type=f32)


def _edge_net0_body(ea_ref, we0_ref, e0lo_ref, e0hi_ref):
    e0 = _dotT(ea_ref[...], we0_ref[...])
    e0lo_ref[...] = e0[:, :64]
    e0hi_ref[...] = e0[:, 64:]


def _edge_net12_body(ea_ref, we1_ref, we2_ref,
                     e1lo_ref, e1hi_ref, e2lo_ref, e2hi_ref):
    ea = ea_ref[...]
    e1 = _dotT(ea, we1_ref[...])
    e1lo_ref[...] = e1[:, :128]
    e1hi_ref[...] = e1[:, 128:]
    e2 = _dotT(ea, we2_ref[...])
    e2lo_ref[...] = e2[:, :128]
    e2hi_ref[...] = e2[:, 128:]


def _edge_net0(edge_attr, We0):
    ne = E // _BE
    return pl.pallas_call(
        _edge_net0_body,
        grid=(ne,),
        in_specs=[
            pl.BlockSpec((_BE, D_E), lambda i: (i, 0)),
            pl.BlockSpec((D_IN, D_E), lambda i: (0, 0)),
        ],
        out_specs=[
            pl.BlockSpec((_BE, 64), lambda i: (i, 0)),
            pl.BlockSpec((_BE, 64), lambda i: (i, 0)),
        ],
        out_shape=[
            jax.ShapeDtypeStruct((E, 64), f32),
            jax.ShapeDtypeStruct((E, 64), f32),
        ],
    )(edge_attr, We0)


def _edge_net12(edge_attr, We1, We2):
    ne = E // _BE
    return pl.pallas_call(
        _edge_net12_body,
        grid=(ne,),
        in_specs=[
            pl.BlockSpec((_BE, D_E), lambda i: (i, 0)),
            pl.BlockSpec((D_H, D_E), lambda i: (0, 0)),
            pl.BlockSpec((D_H, D_E), lambda i: (0, 0)),
        ],
        out_specs=[pl.BlockSpec((_BE, 128), lambda i: (i, 0))] * 4,
        out_shape=[jax.ShapeDtypeStruct((E, 128), f32)] * 4,
    )(edge_attr, We1, We2)


def _update0_body(x_ref, alo_ref, ahi_ref, w_ref, b_ref, lo_ref, hi_ref):
    hsum = x_ref[...] + jnp.concatenate([alo_ref[...], ahi_ref[...]], axis=1)
    y = jnp.maximum(_dotT(hsum, w_ref[...]) + b_ref[...], 0.0)
    lo_ref[...] = y[:, :128]
    hi_ref[...] = y[:, 128:]


def _update0(x, alo, ahi, W0, b0):
    nb = N // _BN
    return pl.pallas_call(
        _update0_body,
        grid=(nb,),
        in_specs=[
            pl.BlockSpec((_BN, D_IN), lambda i: (i, 0)),
            pl.BlockSpec((_BN, 64), lambda i: (i, 0)),
            pl.BlockSpec((_BN, 64), lambda i: (i, 0)),
            pl.BlockSpec((D_H, D_IN), lambda i: (0, 0)),
            pl.BlockSpec((1, D_H), lambda i: (0, 0)),
        ],
        out_specs=[pl.BlockSpec((_BN, 128), lambda i: (i, 0))] * 2,
        out_shape=[jax.ShapeDtypeStruct((N, 128), f32)] * 2,
    )(x, alo, ahi, W0, b0.reshape(1, D_H))


def _update12_body(hlo_ref, hhi_ref, alo_ref, ahi_ref, w_ref, b_ref,
                   lo_ref, hi_ref):
    hsum = jnp.concatenate(
        [hlo_ref[...] + alo_ref[...], hhi_ref[...] + ahi_ref[...]], axis=1)
    y = jnp.maximum(_dotT(hsum, w_ref[...]) + b_ref[...], 0.0)
    lo_ref[...] = y[:, :128]
    hi_ref[...] = y[:, 128:]


def _update12(hlo, hhi, alo, ahi, W, b):
    nb = N // _BN
    return pl.pallas_call(
        _update12_body,
        grid=(nb,),
        in_specs=[pl.BlockSpec((_BN, 128), lambda i: (i, 0))] * 4 + [
            pl.BlockSpec((D_H, D_H), lambda i: (0, 0)),
            pl.BlockSpec((1, D_H), lambda i: (0, 0)),
        ],
        out_specs=[pl.BlockSpec((_BN, 128), lambda i: (i, 0))] * 2,
        out_shape=[jax.ShapeDtypeStruct((N, 128), f32)] * 2,
    )(hlo, hhi, alo, ahi, W, b.reshape(1, D_H))


def _pool_body(batch_ref, h1lo, h1hi, h2lo, h2hi, h3lo, h3hi,
               out_ref, cnt_ref):
    i = pl.program_id(0)
    nb = pl.num_programs(0)
    nr = jnp.concatenate(
        [h1lo[...] + h2lo[...] + h3lo[...],
         h1hi[...] + h2hi[...] + h3hi[...]], axis=1)
    bb = batch_ref[...]
    gids = lax.broadcasted_iota(jnp.int32, (G, _BP), 0)
    mf = jnp.where(bb == gids, 1.0, 0.0).astype(f32)
    s = lax.dot_general(mf, nr, (((1,), (0,)), ((), ())),
                        preferred_element_type=f32)
    c = jnp.sum(mf, axis=1, keepdims=True)

    @pl.when(i == 0)
    def _():
        out_ref[...] = s
        cnt_ref[...] = c

    @pl.when(i > 0)
    def _():
        out_ref[...] = out_ref[...] + s
        cnt_ref[...] = cnt_ref[...] + c

    @pl.when(i == nb - 1)
    def _():
        cnt = cnt_ref[...]
        scale = jnp.where(cnt > 0.5, lax.rsqrt(cnt), 0.0)
        out_ref[...] = out_ref[...] * scale


def _pool(batch2d, h1lo, h1hi, h2lo, h2hi, h3lo, h3hi):
    nb = N // _BP
    return pl.pallas_call(
        _pool_body,
        grid=(nb,),
        in_specs=[pl.BlockSpec((1, _BP), lambda i: (0, i))] + [
            pl.BlockSpec((_BP, 128), lambda i: (i, 0))] * 6,
        out_specs=pl.BlockSpec((G, D_H), lambda i: (0, 0)),
        out_shape=jax.ShapeDtypeStruct((G, D_H), f32),
        scratch_shapes=[pltpu.VMEM((G, 1), f32)],
    )(batch2d, h1lo, h1hi, h2lo, h2hi, h3lo, h3hi)


def kernel(x, edge_index, edge_attr, batch, W0, b0, We0, W1, b1, We1,
           W2, b2, We2):
    src2d = edge_index[0].reshape(E // C_CHUNK, C_CHUNK)
    dst2d = edge_index[1].reshape(E // C_CHUNK, C_CHUNK)

    e0lo, e0hi = _edge_net0(edge_attr, We0)
    e1lo, e1hi, e2lo, e2hi = _edge_net12(edge_attr, We1, We2)

    xlo = x[:, :64]
    xhi = x[:, 64:]
    a0lo, a0hi = _sc_edge_64(xlo, xhi, e0lo, e0hi, src2d, dst2d)
    h1lo, h1hi = _update0(x, a0lo, a0hi, W0, b0)

    a1lo, a1hi = _sc_edge_128(h1lo, h1hi, e1lo, e1hi, src2d, dst2d)
    h2lo, h2hi = _update12(h1lo, h1hi, a1lo, a1hi, W1, b1)

    a2lo, a2hi = _sc_edge_128(h2lo, h2hi, e2lo, e2hi, src2d, dst2d)
    h3lo, h3hi = _update12(h2lo, h2hi, a2lo, a2hi, W2, b2)

    return _pool(batch.reshape(1, N), h1lo, h1hi, h2lo, h2hi, h3lo, h3hi)


# speedup vs baseline: 3.4947x; 3.4947x over previous
import functools

import jax
import jax.numpy as jnp
from jax import lax
from jax.experimental import pallas as pl
from jax.experimental.pallas import tpu as pltpu
from jax.experimental.pallas import tpu_sc as plsc

N = 10000
E = 320000
D_IN = 128
D_H = 256
D_E = 16
G = 128

C_CHUNK = 40
N_TILES = 16
W_ROWS = 624
Z_ROWS = 104

f32 = jnp.float32


GSZ = 20
NCH = E // (N_TILES * C_CHUNK)
NGRP = NCH // GSZ


def _make_sc_edge():
    d_half = 128
    nvec = d_half // 16

    mesh = plsc.VectorSubcoreMesh(core_axis_name="c", subcore_axis_name="s")

    @functools.partial(
        pl.kernel,
        mesh=mesh,
        out_type=(
            jax.ShapeDtypeStruct((N, d_half), f32),
            jax.ShapeDtypeStruct((N, d_half), f32),
        ),
        scratch_types=[
            pltpu.VMEM((2, GSZ, C_CHUNK), jnp.int32),
            pltpu.VMEM((2, GSZ, C_CHUNK), jnp.int32),
            pltpu.VMEM((2, C_CHUNK, d_half), f32),
            pltpu.VMEM((2, C_CHUNK, d_half), f32),
            pltpu.VMEM((2, C_CHUNK, d_half), f32),
            pltpu.VMEM_SHARED((N, d_half), f32),
            pltpu.SemaphoreType.DMA,
            pltpu.SemaphoreType.DMA,
            pltpu.SemaphoreType.DMA,
            pltpu.SemaphoreType.DMA,
            pltpu.SemaphoreType.DMA,
            pltpu.SemaphoreType.DMA,
            pltpu.SemaphoreType.DMA,
            pltpu.SemaphoreType.DMA,
            pltpu.SemaphoreType.DMA,
            pltpu.SemaphoreType.DMA,
        ],
    )
    def sc_edge(h0, h1, e0, e1, src4d, dst4d, out0, out1,
                sidx, didx, rows, ebuf, msg, acc,
                sg0, sg1, se0, se1, ss0, ss1, sis0, sis1, sid0, sid1):
        cid = lax.axis_index("c")
        sid = lax.axis_index("s")
        sg = (sg0, sg1)
        se = (se0, se1)
        ss = (ss0, ss1)
        sis = (sis0, sis1)
        sidd = (sid0, sid1)

        def zfill(r, carry):
            for kv in range(nvec):
                msg[0, r, pl.ds(kv * 16, 16)] = jnp.zeros((16,), f32)
            return carry
        lax.fori_loop(0, C_CHUNK, zfill, 0)

        def run(h, e, out):
            zbase = sid * W_ROWS
            for kz in range(15):
                pltpu.sync_copy(msg.at[0],
                                acc.at[pl.ds(zbase + kz * 40, 40)])
            pltpu.sync_copy(msg.at[0, pl.ds(0, 24)],
                            acc.at[pl.ds(zbase + 600, 24)])

            @pl.when(sid == N_TILES - 1)
            def _():
                pltpu.sync_copy(msg.at[0, pl.ds(0, 16)],
                                acc.at[pl.ds(N_TILES * W_ROWS, 16)])
            plsc.subcore_barrier()

            def idx_start(gslot, g):
                pltpu.make_async_copy(src4d.at[sid, g], sidx.at[gslot],
                                      sis[gslot]).start()
                pltpu.make_async_copy(dst4d.at[sid, g], didx.at[gslot],
                                      sidd[gslot]).start()

            def idx_wait(gslot, g):
                pltpu.make_async_copy(src4d.at[sid, g], sidx.at[gslot],
                                      sis[gslot]).wait()
                pltpu.make_async_copy(dst4d.at[sid, g], didx.at[gslot],
                                      sidd[gslot]).wait()

            def echunk(g, jj):
                return pl.multiple_of(
                    ((sid * NGRP + g) * GSZ + jj) * C_CHUNK, 8)

            def gather_start(slot, gslot, g, jj):
                pltpu.make_async_copy(
                    h.at[sidx.at[gslot, jj]], rows.at[slot], sg[slot]).start()
                pltpu.make_async_copy(
                    e.at[pl.ds(echunk(g, jj), C_CHUNK)],
                    ebuf.at[slot], se[slot]).start()

            def gather_wait(slot, gslot, g, jj):
                pltpu.make_async_copy(
                    h.at[sidx.at[gslot, jj]], rows.at[slot], sg[slot]).wait()
                pltpu.make_async_copy(
                    e.at[pl.ds(echunk(g, jj), C_CHUNK)],
                    ebuf.at[slot], se[slot]).wait()

            def scatter_start(slot, gslot, jj):
                pltpu.make_async_copy(
                    msg.at[slot], acc.at[didx.at[gslot, jj]],
                    ss[slot]).start(add=True)

            def scatter_wait(slot, gslot, jj):
                pltpu.make_async_copy(
                    msg.at[slot], acc.at[didx.at[gslot, jj]],
                    ss[slot]).wait()

            def compute(slot):
                def body(r, carry):
                    for kv in range(nvec):
                        a = rows[slot, r, pl.ds(kv * 16, 16)]
                        bv = ebuf[slot, r, pl.ds(kv * 16, 16)]
                        msg[slot, r, pl.ds(kv * 16, 16)] = (
                            jnp.maximum(a + bv, 0.0))
                    return carry
                lax.fori_loop(0, C_CHUNK, body, 0)

            def do_group(gslot, g):
                gather_start(0, gslot, g, 0)

                def cpair(q, carry):
                    jj0 = 2 * q
                    jj1 = jj0 + 1
                    gather_start(1, gslot, g, jj1)
                    gather_wait(0, gslot, g, jj0)

                    @pl.when(q > 0)
                    def _():
                        scatter_wait(0, gslot, jj0)
                    compute(0)
                    scatter_start(0, gslot, jj0)
                    gather_start(0, gslot, g, jj0 + 2)
                    gather_wait(1, gslot, g, jj1)

                    @pl.when(q > 0)
                    def _():
                        scatter_wait(1, gslot, jj1)
                    compute(1)
                    scatter_start(1, gslot, jj1)
                    return carry

                lax.fori_loop(0, GSZ // 2 - 1, cpair, 0)
                gather_start(1, gslot, g, GSZ - 1)
                gather_wait(0, gslot, g, GSZ - 2)
                scatter_wait(0, gslot, GSZ - 2)
                compute(0)
                scatter_start(0, gslot, GSZ - 2)
                gather_wait(1, gslot, g, GSZ - 1)
                scatter_wait(1, gslot, GSZ - 1)
                compute(1)
                scatter_start(1, gslot, GSZ - 1)
                scatter_wait(0, gslot, GSZ - 2)
                scatter_wait(1, gslot, GSZ - 1)

            idx_start(0, 0)

            def pairg(p, carry):
                g0 = 2 * p
                idx_wait(0, g0)
                idx_start(1, g0 + 1)
                do_group(0, g0)
                idx_wait(1, g0 + 1)
                idx_start(0, g0 + 2)
                do_group(1, g0 + 1)
                return carry

            lax.fori_loop(0, NGRP // 2, pairg, 0)
            idx_wait(0, NGRP - 1)
            do_group(0, NGRP - 1)
            plsc.subcore_barrier()

            pltpu.sync_copy(acc.at[pl.ds(zbase, W_ROWS)],
                            out.at[pl.ds(zbase, W_ROWS)])

            @pl.when(sid == N_TILES - 1)
            def _():
                pltpu.sync_copy(acc.at[pl.ds(N_TILES * W_ROWS, 16)],
                                out.at[pl.ds(N_TILES * W_ROWS, 16)])

        @pl.when(cid == 0)
        def _():
            run(h0, e0, out0)

        @pl.when(cid == 1)
        def _():
            run(h1, e1, out1)

    return sc_edge


_sc_edge_l12 = _make_sc_edge()


_BE = 1000
_BN = 1000
_BP = 2000


def _dotT(a, w):
    return lax.dot_general(a, w, (((1,), (1,)), ((), ())),
                           preferred_element_type=f32)


def _edge_net0_body(ea_ref, we0_ref, e0_ref):
    e0_ref[...] = _dotT(ea_ref[...], we0_ref[...])


def _edge_net12_body(ea_ref, we1_ref, we2_ref,
                     e1lo_ref, e1hi_ref, e2lo_ref, e2hi_ref):
    ea = ea_ref[...]
    e1 = _dotT(ea, we1_ref[...])
    e1lo_ref[...] = e1[:, :128]
    e1hi_ref[...] = e1[:, 128:]
    e2 = _dotT(ea, we2_ref[...])
    e2lo_ref[...] = e2[:, :128]
    e2hi_ref[...] = e2[:, 128:]


def _edge_net0(edge_attr, We0):
    ne = E // _BE
    return pl.pallas_call(
        _edge_net0_body,
        grid=(ne,),
        in_specs=[
            pl.BlockSpec((_BE, D_E), lambda i: (i, 0)),
            pl.BlockSpec((D_IN, D_E), lambda i: (0, 0)),
        ],
        out_specs=pl.BlockSpec((_BE, D_IN), lambda i: (i, 0)),
        out_shape=jax.ShapeDtypeStruct((E, D_IN), f32),
    )(edge_attr, We0)


def _edge_net12(edge_attr, We1, We2):
    ne = E // _BE
    return pl.pallas_call(
        _edge_net12_body,
        grid=(ne,),
        in_specs=[
            pl.BlockSpec((_BE, D_E), lambda i: (i, 0)),
            pl.BlockSpec((D_H, D_E), lambda i: (0, 0)),
            pl.BlockSpec((D_H, D_E), lambda i: (0, 0)),
        ],
        out_specs=[pl.BlockSpec((_BE, 128), lambda i: (i, 0))] * 4,
        out_shape=[jax.ShapeDtypeStruct((E, 128), f32)] * 4,
    )(edge_attr, We1, We2)


def _update0_body(x_ref, a_ref, w_ref, b_ref, lo_ref, hi_ref):
    hsum = x_ref[...] + a_ref[...]
    y = jnp.maximum(_dotT(hsum, w_ref[...]) + b_ref[...], 0.0)
    lo_ref[...] = y[:, :128]
    hi_ref[...] = y[:, 128:]


def _update0(x, a0, W0, b0):
    nb = N // _BN
    return pl.pallas_call(
        _update0_body,
        grid=(nb,),
        in_specs=[pl.BlockSpec((_BN, D_IN), lambda i: (i, 0))] * 2 + [
            pl.BlockSpec((D_H, D_IN), lambda i: (0, 0)),
            pl.BlockSpec((1, D_H), lambda i: (0, 0)),
        ],
        out_specs=[pl.BlockSpec((_BN, 128), lambda i: (i, 0))] * 2,
        out_shape=[jax.ShapeDtypeStruct((N, 128), f32)] * 2,
    )(x, a0, W0, b0.reshape(1, D_H))


def _update12_body(hlo_ref, hhi_ref, alo_ref, ahi_ref, w_ref, b_ref,
                   lo_ref, hi_ref):
    hsum = jnp.concatenate(
        [hlo_ref[...] + alo_ref[...], hhi_ref[...] + ahi_ref[...]], axis=1)
    y = jnp.maximum(_dotT(hsum, w_ref[...]) + b_ref[...], 0.0)
    lo_ref[...] = y[:, :128]
    hi_ref[...] = y[:, 128:]


def _update12(hlo, hhi, alo, ahi, W, b):
    nb = N // _BN
    return pl.pallas_call(
        _update12_body,
        grid=(nb,),
        in_specs=[pl.BlockSpec((_BN, 128), lambda i: (i, 0))] * 4 + [
            pl.BlockSpec((D_H, D_H), lambda i: (0, 0)),
            pl.BlockSpec((1, D_H), lambda i: (0, 0)),
        ],
        out_specs=[pl.BlockSpec((_BN, 128), lambda i: (i, 0))] * 2,
        out_shape=[jax.ShapeDtypeStruct((N, 128), f32)] * 2,
    )(hlo, hhi, alo, ahi, W, b.reshape(1, D_H))


def _pool_body(batch_ref, h1lo, h1hi, h2lo, h2hi, h3lo, h3hi,
               out_ref, cnt_ref):
    i = pl.program_id(0)
    nb = pl.num_programs(0)
    nr = jnp.concatenate(
        [h1lo[...] + h2lo[...] + h3lo[...],
         h1hi[...] + h2hi[...] + h3hi[...]], axis=1)
    bb = batch_ref[...].reshape(1, _BP)
    gids = lax.broadcasted_iota(jnp.int32, (G, _BP), 0)
    mf = jnp.where(bb == gids, 1.0, 0.0).astype(f32)
    s = lax.dot_general(mf, nr, (((1,), (0,)), ((), ())),
                        preferred_element_type=f32)
    c = jnp.sum(mf, axis=1, keepdims=True)

    @pl.when(i == 0)
    def _():
        out_ref[...] = s
        cnt_ref[...] = c

    @pl.when(i > 0)
    def _():
        out_ref[...] = out_ref[...] + s
        cnt_ref[...] = cnt_ref[...] + c

    @pl.when(i == nb - 1)
    def _():
        cnt = cnt_ref[...]
        scale = jnp.where(cnt > 0.5, lax.rsqrt(cnt), 0.0)
        out_ref[...] = out_ref[...] * scale


def _pool(batch3d, h1lo, h1hi, h2lo, h2hi, h3lo, h3hi):
    nb = N // _BP
    return pl.pallas_call(
        _pool_body,
        grid=(nb,),
        in_specs=[pl.BlockSpec((1, 1, _BP), lambda i: (i, 0, 0))] + [
            pl.BlockSpec((_BP, 128), lambda i: (i, 0))] * 6,
        out_specs=pl.BlockSpec((G, D_H), lambda i: (0, 0)),
        out_shape=jax.ShapeDtypeStruct((G, D_H), f32),
        scratch_shapes=[pltpu.VMEM((G, 1), f32)],
    )(batch3d, h1lo, h1hi, h2lo, h2hi, h3lo, h3hi)


def kernel(x, edge_index, edge_attr, batch, W0, b0, We0, W1, b1, We1,
           W2, b2, We2):
    src12 = edge_index[0].reshape(N_TILES, NGRP, GSZ, C_CHUNK)
    dst12 = edge_index[1].reshape(N_TILES, NGRP, GSZ, C_CHUNK)

    e0 = _edge_net0(edge_attr, We0)
    e1lo, e1hi, e2lo, e2hi = _edge_net12(edge_attr, We1, We2)

    a0, _a0dup = _sc_edge_l12(x, x, e0, e0, src12, dst12)
    h1lo, h1hi = _update0(x, a0, W0, b0)

    a1lo, a1hi = _sc_edge_l12(h1lo, h1hi, e1lo, e1hi, src12, dst12)
    h2lo, h2hi = _update12(h1lo, h1hi, a1lo, a1hi, W1, b1)

    a2lo, a2hi = _sc_edge_l12(h2lo, h2hi, e2lo, e2hi, src12, dst12)
    h3lo, h3hi = _update12(h2lo, h2hi, a2lo, a2hi, W2, b2)

    return _pool(batch.reshape(N // _BP, 1, _BP),
                 h1lo, h1hi, h2lo, h2hi, h3lo, h3hi)
